# baseline (speedup 1.0000x reference)
import functools

import jax
import jax.numpy as jnp
from jax.experimental import pallas as pl
from jax.experimental.pallas import tpu as pltpu


def _mm_body(x_ref, w_ref, b_ref, o_ref, *, kb, k_total):
    k = pl.program_id(1)
    nk = pl.num_programs(1)

    @pl.when(k == 0)
    def _init():
        o_ref[...] = jnp.broadcast_to(b_ref[...], o_ref.shape)

    def _acc(x, w):
        o_ref[...] += jax.lax.dot_general(
            x, w,
            dimension_numbers=(((1,), (0,)), ((), ())),
            preferred_element_type=jnp.float32)

    @pl.when(k < nk - 1)
    def _full():
        _acc(x_ref[...], w_ref[...])

    @pl.when(k == nk - 1)
    def _tail():
        x = x_ref[...]
        w = w_ref[...]
        base = k * kb
        xcol = jax.lax.broadcasted_iota(jnp.int32, x.shape, 1) + base
        wrow = jax.lax.broadcasted_iota(jnp.int32, w.shape, 0) + base
        _acc(jnp.where(xcol < k_total, x, 0.0),
             jnp.where(wrow < k_total, w, 0.0))


def kernel(inputs, w, b):
    m, k_total = inputs.shape
    _, n = w.shape
    mb = 512
    kb = 2048
    grid = (m // mb, pl.cdiv(k_total, kb))
    out = pl.pallas_call(
        functools.partial(_mm_body, kb=kb, k_total=k_total),
        grid=grid,
        in_specs=[
            pl.BlockSpec((mb, kb), lambda i, j: (i, j)),
            pl.BlockSpec((kb, n), lambda i, j: (j, 0)),
            pl.BlockSpec((1, n), lambda i, j: (0, 0)),
        ],
        out_specs=pl.BlockSpec((mb, n), lambda i, j: (i, 0)),
        out_shape=jax.ShapeDtypeStruct((m, n), jnp.float32),
        compiler_params=pltpu.CompilerParams(
            dimension_semantics=("parallel", "arbitrary"),
        ),
    )(inputs, w, b.reshape(1, n))
    return out


# speedup vs baseline: 1.0282x; 1.0282x over previous
import functools

import jax
import jax.numpy as jnp
from jax.experimental import pallas as pl
from jax.experimental.pallas import tpu as pltpu


def _mm_body(x_ref, w_ref, b_ref, o_ref, *, kb, k_total):
    k = pl.program_id(1)
    nk = pl.num_programs(1)

    @pl.when(k == 0)
    def _init():
        o_ref[...] = jnp.broadcast_to(b_ref[...], o_ref.shape)

    def _acc(x, w):
        o_ref[...] += jax.lax.dot_general(
            x, w,
            dimension_numbers=(((1,), (0,)), ((), ())),
            preferred_element_type=jnp.float32)

    @pl.when(k < nk - 1)
    def _full():
        _acc(x_ref[...], w_ref[...])

    @pl.when(k == nk - 1)
    def _tail():
        x = x_ref[...]
        w = w_ref[...]
        base = k * kb
        xcol = jax.lax.broadcasted_iota(jnp.int32, x.shape, 1) + base
        wrow = jax.lax.broadcasted_iota(jnp.int32, w.shape, 0) + base
        _acc(jnp.where(xcol < k_total, x, 0.0),
             jnp.where(wrow < k_total, w, 0.0))


def kernel(inputs, w, b):
    m, k_total = inputs.shape
    _, n = w.shape
    mb = 512
    kb = 8192
    grid = (m // mb, pl.cdiv(k_total, kb))
    out = pl.pallas_call(
        functools.partial(_mm_body, kb=kb, k_total=k_total),
        grid=grid,
        in_specs=[
            pl.BlockSpec((mb, kb), lambda i, j: (i, j)),
            pl.BlockSpec((kb, n), lambda i, j: (j, 0)),
            pl.BlockSpec((1, n), lambda i, j: (0, 0)),
        ],
        out_specs=pl.BlockSpec((mb, n), lambda i, j: (i, 0)),
        out_shape=jax.ShapeDtypeStruct((m, n), jnp.float32),
        compiler_params=pltpu.CompilerParams(
            dimension_semantics=("parallel", "arbitrary"),
        ),
    )(inputs, w, b.reshape(1, n))
    return out


# speedup vs baseline: 1.0598x; 1.0306x over previous
import functools

import jax
import jax.numpy as jnp
from jax.experimental import pallas as pl
from jax.experimental.pallas import tpu as pltpu


def _mm_body(x_ref, w_ref, b_ref, o_ref, *, kb, k_total):
    k = pl.program_id(1)
    nk = pl.num_programs(1)

    @pl.when(k == 0)
    def _init():
        o_ref[...] = jnp.broadcast_to(b_ref[...], o_ref.shape)

    def _acc(x, w):
        o_ref[...] += jax.lax.dot_general(
            x, w,
            dimension_numbers=(((1,), (0,)), ((), ())),
            preferred_element_type=jnp.float32)

    @pl.when(k < nk - 1)
    def _full():
        _acc(x_ref[...], w_ref[...])

    @pl.when(k == nk - 1)
    def _tail():
        x = x_ref[...]
        w = w_ref[...]
        base = k * kb
        xcol = jax.lax.broadcasted_iota(jnp.int32, x.shape, 1) + base
        wrow = jax.lax.broadcasted_iota(jnp.int32, w.shape, 0) + base
        _acc(jnp.where(xcol < k_total, x, 0.0),
             jnp.where(wrow < k_total, w, 0.0))


def kernel(inputs, w, b):
    m, k_total = inputs.shape
    _, n = w.shape
    mb = 1024
    kb = 4096
    grid = (m // mb, pl.cdiv(k_total, kb))
    out = pl.pallas_call(
        functools.partial(_mm_body, kb=kb, k_total=k_total),
        grid=grid,
        in_specs=[
            pl.BlockSpec((mb, kb), lambda i, j: (i, j)),
            pl.BlockSpec((kb, n), lambda i, j: (j, 0)),
            pl.BlockSpec((1, n), lambda i, j: (0, 0)),
        ],
        out_specs=pl.BlockSpec((mb, n), lambda i, j: (i, 0)),
        out_shape=jax.ShapeDtypeStruct((m, n), jnp.float32),
        compiler_params=pltpu.CompilerParams(
            dimension_semantics=("parallel", "arbitrary"),
        ),
    )(inputs, w, b.reshape(1, n))
    return out
